# baseline (speedup 1.0000x reference)
import jax
import jax.numpy as jnp
from jax.experimental import pallas as pl
from jax.experimental.pallas import tpu as pltpu

_GROUPS = 4
_BLOCK_I = 256


def _spmm_kernel(xt_ref, *refs):
    w_refs = refs[:_GROUPS]
    o_ref = refs[_GROUPS]
    i = pl.program_id(0)
    n = pl.num_programs(0)
    part = None
    for k, w_ref in enumerate(w_refs):
        base = (k * n + i) * _BLOCK_I
        p = jax.lax.dot_general(
            xt_ref[:, pl.ds(base, _BLOCK_I)], w_ref[...],
            dimension_numbers=(((1,), (0,)), ((), ())),
            preferred_element_type=jnp.float32,
        )
        part = p if part is None else part + p

    @pl.when(i == 0)
    def _():
        o_ref[...] = part

    @pl.when(i > 0)
    def _():
        o_ref[...] += part


def kernel(input, W):
    size_in, cols = input.shape
    size_out = W.shape[1]
    xt = input.T
    n = size_in // (_BLOCK_I * _GROUPS)
    w_specs = [
        pl.BlockSpec((_BLOCK_I, size_out), lambda i, k=k: (k * n + i, 0))
        for k in range(_GROUPS)
    ]
    out_t = pl.pallas_call(
        _spmm_kernel,
        grid=(n,),
        in_specs=[pl.BlockSpec((cols, size_in), lambda i: (0, 0))] + w_specs,
        out_specs=pl.BlockSpec((cols, size_out), lambda i: (0, 0)),
        out_shape=jax.ShapeDtypeStruct((cols, size_out), jnp.float32),
    )(xt, *([W] * _GROUPS))
    return out_t.T
